# baseline (speedup 1.0000x reference)
import jax
import jax.numpy as jnp
from jax.experimental import pallas as pl

_CHUNKS = 2


def _vq_kernel(z_ref, dic_ref, out_ref):
    dic = dic_ref[...]
    k, d = dic.shape
    n = z_ref.shape[0]
    ones = jnp.ones((1, d), jnp.float32)
    d2 = jax.lax.dot_general(
        ones, dic * dic, (((1,), (1,)), ((), ())),
        precision=jax.lax.Precision.HIGHEST,
        preferred_element_type=jnp.float32)
    c = n // _CHUNKS
    for h in range(_CHUNKS):
        z = z_ref[h * c:(h + 1) * c, :]
        cross = jax.lax.dot_general(
            z, dic, (((1,), (1,)), ((), ())),
            precision=jax.lax.Precision.DEFAULT,
            preferred_element_type=jnp.float32)
        z2 = jnp.sum(z * z, axis=1, keepdims=True)
        dist = (d2 + z2) - 2.0 * cross
        minval = jnp.min(dist, axis=1, keepdims=True)
        iota = jax.lax.broadcasted_iota(
            jnp.int32, (c, k), 1).astype(jnp.float32)
        idx = jnp.min(jnp.where(dist == minval, iota, float(k)), axis=1,
                      keepdims=True)
        onehot = (iota == idx).astype(jnp.bfloat16)
        out_ref[h * c:(h + 1) * c, :] = jax.lax.dot_general(
            onehot, dic, (((1,), (0,)), ((), ())),
            precision=jax.lax.Precision.DEFAULT,
            preferred_element_type=jnp.float32)


def kernel(ze, dictionary):
    b, t, d = ze.shape
    n = b * t
    k = dictionary.shape[0]
    z = ze.reshape(n, d)
    out = pl.pallas_call(
        _vq_kernel,
        grid=(1,),
        in_specs=[
            pl.BlockSpec((n, d), lambda i: (0, 0)),
            pl.BlockSpec((k, d), lambda i: (0, 0)),
        ],
        out_specs=pl.BlockSpec((n, d), lambda i: (0, 0)),
        out_shape=jax.ShapeDtypeStruct((n, d), jnp.float32),
    )(z, dictionary)
    return out.reshape(b, t, d)


# speedup vs baseline: 1.0942x; 1.0942x over previous
import jax
import jax.numpy as jnp
from jax.experimental import pallas as pl

_CHUNKS = 4
_G = 128


def _vq_kernel(z_ref, dic_ref, out_ref):
    dic = dic_ref[...]
    k, d = dic.shape
    n = z_ref.shape[0]
    ones = jnp.ones((1, d), jnp.float32)
    d2 = jax.lax.dot_general(
        ones, dic * dic, (((1,), (1,)), ((), ())),
        precision=jax.lax.Precision.HIGHEST,
        preferred_element_type=jnp.float32)
    c = n // _CHUNKS
    for h in range(_CHUNKS):
        z = z_ref[h * c:(h + 1) * c, :]
        cross = jax.lax.dot_general(
            z, dic, (((1,), (1,)), ((), ())),
            precision=jax.lax.Precision.DEFAULT,
            preferred_element_type=jnp.float32)
        z2 = jnp.sum(z * z, axis=1, keepdims=True)
        giota = jax.lax.broadcasted_iota(
            jnp.int32, (c, _G), 1).astype(jnp.float32)
        vacc = (d2[:, :_G] + z2) - 2.0 * cross[:, :_G]
        iacc = giota
        for g in range(1, k // _G):
            dist_g = (d2[:, g * _G:(g + 1) * _G] + z2) \
                - 2.0 * cross[:, g * _G:(g + 1) * _G]
            lt = dist_g < vacc
            iacc = jnp.where(lt, giota + float(g * _G), iacc)
            vacc = jnp.minimum(vacc, dist_g)
        minval = jnp.min(vacc, axis=1, keepdims=True)
        idx = jnp.min(jnp.where(vacc == minval, iacc, float(k)), axis=1,
                      keepdims=True)
        iota = jax.lax.broadcasted_iota(
            jnp.int32, (c, k), 1).astype(jnp.float32)
        onehot = (iota == idx).astype(jnp.bfloat16)
        out_ref[h * c:(h + 1) * c, :] = jax.lax.dot_general(
            onehot, dic, (((1,), (0,)), ((), ())),
            precision=jax.lax.Precision.DEFAULT,
            preferred_element_type=jnp.float32)


def kernel(ze, dictionary):
    b, t, d = ze.shape
    n = b * t
    k = dictionary.shape[0]
    z = ze.reshape(n, d)
    out = pl.pallas_call(
        _vq_kernel,
        grid=(1,),
        in_specs=[
            pl.BlockSpec((n, d), lambda i: (0, 0)),
            pl.BlockSpec((k, d), lambda i: (0, 0)),
        ],
        out_specs=pl.BlockSpec((n, d), lambda i: (0, 0)),
        out_shape=jax.ShapeDtypeStruct((n, d), jnp.float32),
    )(z, dictionary)
    return out.reshape(b, t, d)
